# Initial kernel scaffold; baseline (speedup 1.0000x reference)
import functools

import jax
import jax.numpy as jnp
from jax import lax
from jax.experimental import pallas as pl
from jax.experimental.pallas import tpu as pltpu
from jax.experimental.pallas import tpu_sc as plsc

_N = 10000
_E = 160000
_D = 128

_NC = 2
_NS = 16
_NW = _NC * _NS
_RPT = _N // _NS
_CH = 128
_NCHUNK = _E // _CH
_CPW = (_NCHUNK + _NW - 1) // _NW
_DEGW = 16

_sc_mesh = functools.partial(
    plsc.VectorSubcoreMesh, core_axis_name="c", subcore_axis_name="s")


def _deg_partials(dst):
    @functools.partial(
        pl.kernel,
        mesh=_sc_mesh(),
        out_type=jax.ShapeDtypeStruct((_NC, _N, _DEGW), jnp.float32),
        scratch_types=[
            pltpu.VMEM((1, _CH), jnp.int32),
            pltpu.VMEM((_CH, _DEGW), jnp.float32),
            pltpu.VMEM((_RPT, _DEGW), jnp.float32),
            pltpu.VMEM_SHARED((_N, _DEGW), jnp.float32),
            pltpu.SemaphoreType.DMA,
        ],
    )
    def k(dst_hbm, out_hbm, didx, ones_v, zbuf, acc, sem):
        c = lax.axis_index("c")
        s = lax.axis_index("s")
        wid = s * _NC + c

        def fill(i, carry):
            @pl.when(i < _CH)
            def _():
                ones_v[i] = jnp.ones((_DEGW,), jnp.float32)
            zbuf[i] = jnp.zeros((_DEGW,), jnp.float32)
            return carry

        lax.fori_loop(0, _RPT, fill, 0)
        pltpu.sync_copy(zbuf, acc.at[pl.ds(s * _RPT, _RPT)])
        plsc.subcore_barrier()

        def body(j, carry):
            ch = wid + _NW * j

            @pl.when(ch < _NCHUNK)
            def _():
                pltpu.sync_copy(dst_hbm.at[pl.ds(ch * _CH, _CH)], didx.at[0])
                pltpu.sync_copy(ones_v, acc.at[didx.at[0]], add=True)

            return carry

        lax.fori_loop(0, _CPW, body, 0)
        plsc.subcore_barrier()
        pltpu.sync_copy(acc.at[pl.ds(s * _RPT, _RPT)], zbuf)
        pltpu.sync_copy(zbuf, out_hbm.at[c, pl.ds(s * _RPT, _RPT)])

    return k(dst)


def _msg_partials(src, dst, g):
    @functools.partial(
        pl.kernel,
        mesh=_sc_mesh(),
        out_type=jax.ShapeDtypeStruct((_NC, _N, _D), jnp.float32),
        scratch_types=[
            pltpu.VMEM((_CH,), jnp.int32),
            pltpu.VMEM((1, _CH), jnp.int32),
            pltpu.VMEM((_CH, _D), jnp.float32),
            pltpu.VMEM((_RPT, _D), jnp.float32),
            pltpu.VMEM_SHARED((_N, _D), jnp.float32),
            pltpu.SemaphoreType.DMA,
        ],
    )
    def k(src_hbm, dst_hbm, g_hbm, out_hbm, sidx, didx, rows, zbuf, acc, sem):
        c = lax.axis_index("c")
        s = lax.axis_index("s")
        wid = s * _NC + c

        def fill(i, carry):
            for jj in range(_D // 16):
                zbuf[i, pl.ds(jj * 16, 16)] = jnp.zeros((16,), jnp.float32)
            return carry

        lax.fori_loop(0, _RPT, fill, 0)
        pltpu.sync_copy(zbuf, acc.at[pl.ds(s * _RPT, _RPT)])
        plsc.subcore_barrier()

        def body(j, carry):
            ch = wid + _NW * j

            @pl.when(ch < _NCHUNK)
            def _():
                base = ch * _CH
                pltpu.sync_copy(src_hbm.at[pl.ds(base, _CH)], sidx)
                pltpu.sync_copy(dst_hbm.at[pl.ds(base, _CH)], didx.at[0])
                pltpu.async_copy(g_hbm.at[sidx], rows, sem).wait()
                pltpu.sync_copy(rows, acc.at[didx.at[0]], add=True)

            return carry

        lax.fori_loop(0, _CPW, body, 0)
        plsc.subcore_barrier()
        pltpu.sync_copy(acc.at[pl.ds(s * _RPT, _RPT)], zbuf)
        pltpu.sync_copy(zbuf, out_hbm.at[c, pl.ds(s * _RPT, _RPT)])

    return k(src, dst, g)


_BM = 2000
_BK = 2000


def _prep_kernel(x_ref, wc_ref, wh_ref, degp_ref, g_ref, h_ref):
    deg = degp_ref[0, :, 0] + degp_ref[1, :, 0] + 1.0
    dis = lax.rsqrt(deg)
    xb = x_ref[...]
    hc = lax.dot_general(xb, wc_ref[...], (((1,), (1,)), ((), ())),
                         preferred_element_type=jnp.float32)
    g_ref[...] = hc * dis[:, None]
    hh = lax.dot_general(xb, wh_ref[...], (((1,), (1,)), ((), ())),
                         preferred_element_type=jnp.float32)
    h_ref[...] = jnp.maximum(hh, 0.0)


def _prep(x, W_conv, W_high, degp):
    grid = (_N // _BM,)
    return pl.pallas_call(
        _prep_kernel,
        grid=grid,
        in_specs=[
            pl.BlockSpec((_BM, _D), lambda i: (i, 0)),
            pl.BlockSpec((_D, _D), lambda i: (0, 0)),
            pl.BlockSpec((_D, _D), lambda i: (0, 0)),
            pl.BlockSpec((_NC, _BM, _DEGW), lambda i: (0, i, 0)),
        ],
        out_specs=[
            pl.BlockSpec((_BM, _D), lambda i: (i, 0)),
            pl.BlockSpec((_BM, _D), lambda i: (i, 0)),
        ],
        out_shape=[
            jax.ShapeDtypeStruct((_N, _D), jnp.float32),
            jax.ShapeDtypeStruct((_N, _D), jnp.float32),
        ],
    )(x, W_conv, W_high, degp)


def _mm_kernel(a_ref, b_ref, o_ref):
    kk = pl.program_id(1)
    part = jnp.dot(a_ref[...], b_ref[...], preferred_element_type=jnp.float32)

    @pl.when(kk == 0)
    def _():
        o_ref[...] = part

    @pl.when(kk != 0)
    def _():
        o_ref[...] += part


def _mm(a, b):
    grid = (_N // _BM, _N // _BK)
    return pl.pallas_call(
        _mm_kernel,
        grid=grid,
        in_specs=[
            pl.BlockSpec((_BM, _BK), lambda i, k: (i, k)),
            pl.BlockSpec((_BK, _D), lambda i, k: (k, 0)),
        ],
        out_specs=pl.BlockSpec((_BM, _D), lambda i, k: (i, 0)),
        out_shape=jax.ShapeDtypeStruct((_N, _D), jnp.float32),
        compiler_params=pltpu.CompilerParams(
            dimension_semantics=("arbitrary", "arbitrary")),
    )(a, b)


def _combine_kernel(t3_ref, sp_ref, g_ref, degp_ref, b_ref, sc_ref, o_ref):
    deg = degp_ref[0, :, 0] + degp_ref[1, :, 0] + 1.0
    dis = lax.rsqrt(deg)
    S = sp_ref[0] + sp_ref[1] + g_ref[...]
    gcn = S * dis[:, None] + b_ref[...]
    hl = jnp.maximum(gcn, 0.0)
    o_ref[...] = sc_ref[0, 0] * hl + sc_ref[0, 1] * t3_ref[...]


def _combine(t3, Sp, g, degp, b, scal):
    grid = (_N // _BM,)
    return pl.pallas_call(
        _combine_kernel,
        grid=grid,
        in_specs=[
            pl.BlockSpec((_BM, _D), lambda i: (i, 0)),
            pl.BlockSpec((_NC, _BM, _D), lambda i: (0, i, 0)),
            pl.BlockSpec((_BM, _D), lambda i: (i, 0)),
            pl.BlockSpec((_NC, _BM, _DEGW), lambda i: (0, i, 0)),
            pl.BlockSpec((1, _D), lambda i: (0, 0)),
            pl.BlockSpec((1, 2), lambda i: (0, 0)),
        ],
        out_specs=pl.BlockSpec((_BM, _D), lambda i: (i, 0)),
        out_shape=jax.ShapeDtypeStruct((_N, _D), jnp.float32),
    )(t3, Sp, g, degp, b, scal)


def kernel(x, edge_index, lap, d_inv, W_high, W_conv, b_conv, aL, aH):
    src = edge_index[0]
    dst = edge_index[1]
    degp = _deg_partials(dst)
    g, H = _prep(x, W_conv, W_high, degp)
    Sp = _msg_partials(src, dst, g)
    t1 = _mm(d_inv, H)
    t2 = _mm(lap, t1)
    t3 = _mm(d_inv, t2)
    scal = jnp.concatenate([aL, aH]).reshape(1, 2)
    return _combine(t3, Sp, g, degp, b_conv.reshape(1, _D), scal)


# speedup vs baseline: 13.8935x; 13.8935x over previous
import functools

import jax
import jax.numpy as jnp
from jax import lax
from jax.experimental import pallas as pl
from jax.experimental.pallas import tpu as pltpu
from jax.experimental.pallas import tpu_sc as plsc

_N = 10000
_E = 160000
_D = 128

_NC = 2
_NS = 16
_HALF = _N // _NC
_TRASH = _HALF
_ACCR = _HALF + 8
_RP = 312
_RTAIL = _HALF - _RP * _NS
_CH = 128
_NCHUNK = _E // _CH
_CPS = (_NCHUNK + _NS - 1) // _NS
_DEGW = 16

_sc_mesh = functools.partial(
    plsc.VectorSubcoreMesh, core_axis_name="c", subcore_axis_name="s")


def _localize(didx, c):
    lo = c * _HALF
    for kk in range(_CH // 16):
        v = didx[0, pl.ds(kk * 16, 16)]
        vl = v - lo
        inb = jnp.logical_and(vl >= 0, vl < _HALF)
        didx[0, pl.ds(kk * 16, 16)] = jnp.where(inb, vl, _TRASH)


def _deg_counts(dst):
    @functools.partial(
        pl.kernel,
        mesh=_sc_mesh(),
        out_type=jax.ShapeDtypeStruct((_N, _DEGW), jnp.float32),
        scratch_types=[
            pltpu.VMEM((1, _CH), jnp.int32),
            pltpu.VMEM((_CH, _DEGW), jnp.float32),
            pltpu.VMEM((_RP, _DEGW), jnp.float32),
            pltpu.VMEM_SHARED((_ACCR, _DEGW), jnp.float32),
            pltpu.SemaphoreType.DMA,
        ],
    )
    def k(dst_hbm, out_hbm, didx, ones_v, zbuf, acc, sem):
        c = lax.axis_index("c")
        s = lax.axis_index("s")

        def fill(i, carry):
            @pl.when(i < _CH)
            def _():
                ones_v[i] = jnp.ones((_DEGW,), jnp.float32)
            zbuf[i] = jnp.zeros((_DEGW,), jnp.float32)
            return carry

        lax.fori_loop(0, _RP, fill, 0)
        pltpu.sync_copy(zbuf, acc.at[pl.ds(s * _RP, _RP)])

        @pl.when(s == 0)
        def _():
            pltpu.sync_copy(zbuf.at[pl.ds(0, _ACCR - _RP * _NS)],
                            acc.at[pl.ds(_RP * _NS, _ACCR - _RP * _NS)])

        plsc.subcore_barrier()

        def body(j, carry):
            ch = s + _NS * j

            @pl.when(ch < _NCHUNK)
            def _():
                pltpu.sync_copy(dst_hbm.at[pl.ds(ch * _CH, _CH)], didx.at[0])
                _localize(didx, c)
                pltpu.sync_copy(ones_v, acc.at[didx.at[0]], add=True)

            return carry

        lax.fori_loop(0, _CPS, body, 0)
        plsc.subcore_barrier()
        pltpu.sync_copy(acc.at[pl.ds(s * _RP, _RP)], zbuf)
        pltpu.sync_copy(zbuf, out_hbm.at[pl.ds(c * _HALF + s * _RP, _RP)])

        @pl.when(s == 0)
        def _():
            pltpu.sync_copy(acc.at[pl.ds(_RP * _NS, _RTAIL)],
                            zbuf.at[pl.ds(0, _RTAIL)])
            pltpu.sync_copy(zbuf.at[pl.ds(0, _RTAIL)],
                            out_hbm.at[pl.ds(c * _HALF + _RP * _NS, _RTAIL)])

    return k(dst)


def _msg_sums(src, dst, g):
    @functools.partial(
        pl.kernel,
        mesh=_sc_mesh(),
        out_type=jax.ShapeDtypeStruct((_N, _D), jnp.float32),
        scratch_types=[
            pltpu.VMEM((_CH,), jnp.int32),
            pltpu.VMEM((1, _CH), jnp.int32),
            pltpu.VMEM((_CH, _D), jnp.float32),
            pltpu.VMEM((_RP, _D), jnp.float32),
            pltpu.VMEM_SHARED((_ACCR, _D), jnp.float32),
            pltpu.SemaphoreType.DMA,
        ],
    )
    def k(src_hbm, dst_hbm, g_hbm, out_hbm, sidx, didx, rows, zbuf, acc, sem):
        c = lax.axis_index("c")
        s = lax.axis_index("s")

        def fill(i, carry):
            for jj in range(_D // 16):
                zbuf[i, pl.ds(jj * 16, 16)] = jnp.zeros((16,), jnp.float32)
            return carry

        lax.fori_loop(0, _RP, fill, 0)
        pltpu.sync_copy(zbuf, acc.at[pl.ds(s * _RP, _RP)])

        @pl.when(s == 0)
        def _():
            pltpu.sync_copy(zbuf.at[pl.ds(0, _ACCR - _RP * _NS)],
                            acc.at[pl.ds(_RP * _NS, _ACCR - _RP * _NS)])

        plsc.subcore_barrier()

        def body(j, carry):
            ch = s + _NS * j

            @pl.when(ch < _NCHUNK)
            def _():
                base = ch * _CH
                pltpu.sync_copy(src_hbm.at[pl.ds(base, _CH)], sidx)
                pltpu.sync_copy(dst_hbm.at[pl.ds(base, _CH)], didx.at[0])
                _localize(didx, c)
                pltpu.async_copy(g_hbm.at[sidx], rows, sem).wait()
                pltpu.sync_copy(rows, acc.at[didx.at[0]], add=True)

            return carry

        lax.fori_loop(0, _CPS, body, 0)
        plsc.subcore_barrier()
        pltpu.sync_copy(acc.at[pl.ds(s * _RP, _RP)], zbuf)
        pltpu.sync_copy(zbuf, out_hbm.at[pl.ds(c * _HALF + s * _RP, _RP)])

        @pl.when(s == 0)
        def _():
            pltpu.sync_copy(acc.at[pl.ds(_RP * _NS, _RTAIL)],
                            zbuf.at[pl.ds(0, _RTAIL)])
            pltpu.sync_copy(zbuf.at[pl.ds(0, _RTAIL)],
                            out_hbm.at[pl.ds(c * _HALF + _RP * _NS, _RTAIL)])

    return k(src, dst, g)


_BM = 2000
_BMM = 400


def _prep_kernel(x_ref, wc_ref, wh_ref, degc_ref, g_ref, h_ref):
    deg = degc_ref[:, 0] + 1.0
    dis = lax.rsqrt(deg)
    xb = x_ref[...]
    hc = lax.dot_general(xb, wc_ref[...], (((1,), (1,)), ((), ())),
                         preferred_element_type=jnp.float32)
    g_ref[...] = hc * dis[:, None]
    hh = lax.dot_general(xb, wh_ref[...], (((1,), (1,)), ((), ())),
                         preferred_element_type=jnp.float32)
    h_ref[...] = jnp.maximum(hh, 0.0)


def _prep(x, W_conv, W_high, degc):
    grid = (_N // _BM,)
    return pl.pallas_call(
        _prep_kernel,
        grid=grid,
        in_specs=[
            pl.BlockSpec((_BM, _D), lambda i: (i, 0)),
            pl.BlockSpec((_D, _D), lambda i: (0, 0)),
            pl.BlockSpec((_D, _D), lambda i: (0, 0)),
            pl.BlockSpec((_BM, _DEGW), lambda i: (i, 0)),
        ],
        out_specs=[
            pl.BlockSpec((_BM, _D), lambda i: (i, 0)),
            pl.BlockSpec((_BM, _D), lambda i: (i, 0)),
        ],
        out_shape=[
            jax.ShapeDtypeStruct((_N, _D), jnp.float32),
            jax.ShapeDtypeStruct((_N, _D), jnp.float32),
        ],
    )(x, W_conv, W_high, degc)


def _mm_kernel(a_ref, b_ref, o_ref):
    o_ref[...] = jnp.dot(a_ref[...], b_ref[...],
                         preferred_element_type=jnp.float32)


def _mm(a, b):
    grid = (_N // _BMM,)
    return pl.pallas_call(
        _mm_kernel,
        grid=grid,
        in_specs=[
            pl.BlockSpec((_BMM, _N), lambda i: (i, 0)),
            pl.BlockSpec((_N, _D), lambda i: (0, 0)),
        ],
        out_specs=pl.BlockSpec((_BMM, _D), lambda i: (i, 0)),
        out_shape=jax.ShapeDtypeStruct((_N, _D), jnp.float32),
        compiler_params=pltpu.CompilerParams(
            dimension_semantics=("arbitrary",)),
    )(a, b)


def _combine_kernel(t3_ref, sp_ref, g_ref, degc_ref, b_ref, sc_ref, o_ref):
    deg = degc_ref[:, 0] + 1.0
    dis = lax.rsqrt(deg)
    S = sp_ref[...] + g_ref[...]
    gcn = S * dis[:, None] + b_ref[...]
    hl = jnp.maximum(gcn, 0.0)
    o_ref[...] = sc_ref[0, 0] * hl + sc_ref[0, 1] * t3_ref[...]


def _combine(t3, Sp, g, degc, b, scal):
    grid = (_N // _BM,)
    return pl.pallas_call(
        _combine_kernel,
        grid=grid,
        in_specs=[
            pl.BlockSpec((_BM, _D), lambda i: (i, 0)),
            pl.BlockSpec((_BM, _D), lambda i: (i, 0)),
            pl.BlockSpec((_BM, _D), lambda i: (i, 0)),
            pl.BlockSpec((_BM, _DEGW), lambda i: (i, 0)),
            pl.BlockSpec((1, _D), lambda i: (0, 0)),
            pl.BlockSpec((1, 2), lambda i: (0, 0)),
        ],
        out_specs=pl.BlockSpec((_BM, _D), lambda i: (i, 0)),
        out_shape=jax.ShapeDtypeStruct((_N, _D), jnp.float32),
    )(t3, Sp, g, degc, b, scal)


def kernel(x, edge_index, lap, d_inv, W_high, W_conv, b_conv, aL, aH):
    src = edge_index[0]
    dst = edge_index[1]
    degc = _deg_counts(dst)
    g, H = _prep(x, W_conv, W_high, degc)
    Sp = _msg_sums(src, dst, g)
    t1 = _mm(d_inv, H)
    t2 = _mm(lap, t1)
    t3 = _mm(d_inv, t2)
    scal = jnp.concatenate([aL, aH]).reshape(1, 2)
    return _combine(t3, Sp, g, degc, b_conv.reshape(1, _D), scal)
